# baseline (speedup 1.0000x reference)
import functools

import jax
import jax.numpy as jnp
from jax import lax
from jax.experimental import pallas as pl
from jax.experimental.pallas import tpu as pltpu
from jax.experimental.pallas import tpu_sc as plsc

_NUM_CLASSES = 7
_DIM = 512
_NC = 2
_NS = 16
_NW = _NC * _NS


def _scale_body(t_ref, o_ref):
    o_ref[...] = t_ref[...] * (1.0 / _NUM_CLASSES)


@functools.cache
def _make_gather(B):
    b_per_w = B // _NW
    mesh = plsc.VectorSubcoreMesh(core_axis_name="c", subcore_axis_name="s")

    @functools.partial(
        pl.kernel,
        out_type=jax.ShapeDtypeStruct((B, _DIM), jnp.float32),
        mesh=mesh,
        scratch_types=[
            pltpu.VMEM((b_per_w,), jnp.int32),
            pltpu.VMEM((b_per_w, _DIM), jnp.float32),
            pltpu.SemaphoreType.DMA,
        ],
    )
    def gather_kernel(table_hbm, idx_hbm, out_hbm, idx_v, rows_v, sem):
        wid = lax.axis_index("s") * _NC + lax.axis_index("c")
        base = wid * b_per_w
        pltpu.sync_copy(idx_hbm.at[pl.ds(base, b_per_w)], idx_v)
        pltpu.async_copy(table_hbm.at[idx_v], rows_v, sem).wait()
        pltpu.sync_copy(rows_v, out_hbm.at[pl.ds(base, b_per_w)])

    return gather_kernel


def kernel(x, de_class, clip_prompt):
    B = x.shape[0]
    scaled = pl.pallas_call(
        _scale_body,
        out_shape=jax.ShapeDtypeStruct((_NUM_CLASSES, _DIM), jnp.float32),
    )(clip_prompt)
    idx = de_class.astype(jnp.int32)
    return _make_gather(B)(scaled, idx)
